# baseline (speedup 1.0000x reference)
import jax
import jax.numpy as jnp
from jax import lax
from jax.experimental import pallas as pl
from jax.experimental.pallas import tpu as pltpu
from jax.experimental.pallas import tpu_sc as plsc

N_NODES = 10000
D_FEAT = 128
N_EDGES = 320000

_NC = 2
_NS = 16
_NW = _NC * _NS
_EPT = N_EDGES // _NW
_K = 80
_NPASS = 5
_EPP = _EPT // _NPASS
_CPP = _EPP // _K
_N_PAD = 10240
_RPT = _N_PAD // _NS


def _scale_chunk(rows16, rowsf, vals_p, ci):
    def gbody(g, _):
        vv = vals_p[pl.ds(ci * _K + g * 16, 16)]
        blk16 = rows16.at[pl.ds(g * 16, 16)]
        blkf = rowsf.at[pl.ds(g * 16, 16)]
        for j in range(16):
            vs = jnp.broadcast_to(vv[j], (16,))
            for d in range(D_FEAT // 32):
                packed = blk16[j, pl.ds(d * 16, 16)]
                pair = plsc.bitcast(packed, jnp.bfloat16)
                lo, hi = plsc.unpack(pair, format=plsc.PackFormat.INTERLEAVED)
                blkf[j, pl.ds(d * 32, 16)] = lo * vs
                blkf[j, pl.ds(d * 32 + 16, 16)] = hi * vs
        return 0

    lax.fori_loop(0, _K // 16, gbody, 0)


def _sc_body(x16_hbm, src_hbm, tgt_hbm, vals_hbm, out_hbm,
             acc_sh, r16_a, r16_b, rf_a, rf_b, src_p, tgt_p, vals_p,
             sem_a, sem_b, sem_sa, sem_sb):
    c = lax.axis_index("c")
    s = lax.axis_index("s")
    wid = s * _NC + c

    zeros16 = jnp.zeros((16,), jnp.float32)

    def zbody(i, _):
        for d in range(D_FEAT // 16):
            rf_a[i, pl.ds(d * 16, 16)] = zeros16
        return 0

    lax.fori_loop(0, _K, zbody, 0)
    r0 = s * _RPT
    for j in range(_RPT // _K):
        pltpu.sync_copy(rf_a, acc_sh.at[pl.ds(r0 + j * _K, _K)])
    plsc.subcore_barrier()

    def gather_start(ci, rows, sem):
        pltpu.async_copy(x16_hbm.at[src_p.at[pl.ds(ci * _K, _K)]], rows, sem)

    def gather_wait(ci, rows, sem):
        pltpu.make_async_copy(
            x16_hbm.at[src_p.at[pl.ds(ci * _K, _K)]], rows, sem).wait()

    def scat_fire(ci, rowsf, sem):
        for g in range(_K // 16):
            tv = tgt_p[pl.ds(ci * _K + g * 16, 16)]
            pltpu.async_copy(rowsf.at[pl.ds(g * 16, 16)], acc_sh.at[tv], sem,
                             add=True)

    def scat_drain(ci, rowsf, sem):
        for g in range(_K // 16):
            tv = tgt_p[pl.ds(ci * _K + g * 16, 16)]
            pltpu.make_async_copy(rowsf.at[pl.ds(g * 16, 16)], acc_sh.at[tv],
                                  sem).wait()

    for ps in range(_NPASS):
        e0 = wid * _EPT + ps * _EPP
        pltpu.sync_copy(src_hbm.at[pl.ds(e0, _EPP)], src_p)
        pltpu.sync_copy(tgt_hbm.at[pl.ds(e0, _EPP)], tgt_p)
        pltpu.sync_copy(vals_hbm.at[pl.ds(e0, _EPP)], vals_p)

        gather_start(0, r16_a, sem_a)

        def pbody(p, _):
            ci0 = 2 * p
            gather_wait(ci0, r16_a, sem_a)
            gather_start(ci0 + 1, r16_b, sem_b)

            @pl.when(p > 0)
            def _():
                scat_drain(ci0 - 1, rf_b, sem_sb)

            _scale_chunk(r16_a, rf_a, vals_p, ci0)
            scat_fire(ci0, rf_a, sem_sa)
            gather_wait(ci0 + 1, r16_b, sem_b)
            gather_start(ci0 + 2, r16_a, sem_a)
            scat_drain(ci0, rf_a, sem_sa)
            _scale_chunk(r16_b, rf_b, vals_p, ci0 + 1)
            scat_fire(ci0 + 1, rf_b, sem_sb)
            return 0

        lax.fori_loop(0, (_CPP - 1) // 2, pbody, 0)
        gather_wait(_CPP - 1, r16_a, sem_a)
        scat_drain(_CPP - 2, rf_b, sem_sb)
        _scale_chunk(r16_a, rf_a, vals_p, _CPP - 1)
        scat_fire(_CPP - 1, rf_a, sem_sa)
        scat_drain(_CPP - 1, rf_a, sem_sa)

    plsc.subcore_barrier()

    for j in range(_RPT // _K):
        rr = r0 + j * _K
        pltpu.async_copy(acc_sh.at[pl.ds(rr, _K)],
                         out_hbm.at[c].at[pl.ds(rr, _K)], sem_a)
    for j in range(_RPT // _K):
        rr = r0 + j * _K
        pltpu.make_async_copy(acc_sh.at[pl.ds(rr, _K)],
                              out_hbm.at[c].at[pl.ds(rr, _K)], sem_a).wait()


def _tc_add_body(a_ref, b_ref, o_ref):
    o_ref[...] = a_ref[...] + b_ref[...]


def kernel(x_source, neighborhood_indices, neighborhood_values):
    tgt = neighborhood_indices[0]
    src = neighborhood_indices[1]
    x16 = jax.lax.bitcast_convert_type(
        x_source.astype(jnp.bfloat16)
        .reshape(N_NODES, D_FEAT // 32, 2, 16)
        .transpose(0, 1, 3, 2)
        .reshape(N_NODES, D_FEAT // 2, 2),
        jnp.int32)

    mesh = plsc.VectorSubcoreMesh(core_axis_name="c", subcore_axis_name="s")
    partials = pl.kernel(
        _sc_body,
        mesh=mesh,
        compiler_params=pltpu.CompilerParams(needs_layout_passes=False,
                                             use_tc_tiling_on_sc=False),
        out_type=jax.ShapeDtypeStruct((_NC, _N_PAD, D_FEAT), jnp.float32),
        scratch_types=[
            pltpu.VMEM_SHARED((_N_PAD, D_FEAT), jnp.float32),
            pltpu.VMEM((_K, D_FEAT // 2), jnp.int32),
            pltpu.VMEM((_K, D_FEAT // 2), jnp.int32),
            pltpu.VMEM((_K, D_FEAT), jnp.float32),
            pltpu.VMEM((_K, D_FEAT), jnp.float32),
            pltpu.VMEM((_EPP,), jnp.int32),
            pltpu.VMEM((_EPP,), jnp.int32),
            pltpu.VMEM((_EPP,), jnp.float32),
            pltpu.SemaphoreType.DMA,
            pltpu.SemaphoreType.DMA,
            pltpu.SemaphoreType.DMA,
            pltpu.SemaphoreType.DMA,
        ],
    )(x16, src, tgt, neighborhood_values)

    blk = 1000
    out = pl.pallas_call(
        _tc_add_body,
        out_shape=jax.ShapeDtypeStruct((N_NODES, D_FEAT), jnp.float32),
        grid=(N_NODES // blk,),
        in_specs=[
            pl.BlockSpec((blk, D_FEAT), lambda i: (i, 0)),
            pl.BlockSpec((blk, D_FEAT), lambda i: (i, 0)),
        ],
        out_specs=pl.BlockSpec((blk, D_FEAT), lambda i: (i, 0)),
    )(partials[0], partials[1])
    return out


# speedup vs baseline: 1.9843x; 1.9843x over previous
import jax
import jax.numpy as jnp
from jax import lax
from jax.experimental import pallas as pl
from jax.experimental.pallas import tpu as pltpu
from jax.experimental.pallas import tpu_sc as plsc

N_NODES = 10000
D_FEAT = 128
N_EDGES = 320000

_NC = 2
_NS = 16
_NW = _NC * _NS
_EPT = N_EDGES // _NW
_K = 80
_NPASS = 5
_EPP = _EPT // _NPASS
_CPP = _EPP // _K
_NBUF = 4
_N_PAD = 10240
_RPT = _N_PAD // _NS


def _scale_chunk(rows, vals_p, ci):
    def gbody(g, _):
        vv = vals_p[pl.ds(ci * _K + g * 16, 16)]
        blk = rows.at[pl.ds(g * 16, 16)]
        for j in range(16):
            vs = jnp.broadcast_to(vv[j], (16,))
            for d in range(D_FEAT // 16):
                sl = pl.ds(d * 16, 16)
                blk[j, sl] = blk[j, sl] * vs
        return 0

    lax.fori_loop(0, _K // 16, gbody, 0)


def _sc_body(x_hbm, src_hbm, tgt_hbm, vals_hbm, out_hbm,
             acc_sh, b0, b1, b2, b3, src_p, tgt_p, vals_p,
             g0, g1, g2, g3, s0, s1, s2, s3):
    bufs = (b0, b1, b2, b3)
    gsems = (g0, g1, g2, g3)
    ssems = (s0, s1, s2, s3)
    c = lax.axis_index("c")
    s = lax.axis_index("s")
    wid = s * _NC + c

    zeros16 = jnp.zeros((16,), jnp.float32)

    def zbody(i, _):
        for d in range(D_FEAT // 16):
            b0[i, pl.ds(d * 16, 16)] = zeros16
        return 0

    lax.fori_loop(0, _K, zbody, 0)
    r0 = s * _RPT
    for j in range(_RPT // _K):
        pltpu.sync_copy(b0, acc_sh.at[pl.ds(r0 + j * _K, _K)])
    plsc.subcore_barrier()

    def gather_start(ci, b):
        pltpu.async_copy(x_hbm.at[src_p.at[pl.ds(ci * _K, _K)]],
                         bufs[b], gsems[b])

    def gather_wait(ci, b):
        pltpu.make_async_copy(x_hbm.at[src_p.at[pl.ds(ci * _K, _K)]],
                              bufs[b], gsems[b]).wait()

    def scat_fire(ci, b):
        for g in range(_K // 16):
            tv = tgt_p[pl.ds(ci * _K + g * 16, 16)]
            pltpu.async_copy(bufs[b].at[pl.ds(g * 16, 16)], acc_sh.at[tv],
                             ssems[b], add=True)

    def scat_drain(ci, b):
        for g in range(_K // 16):
            tv = tgt_p[pl.ds(ci * _K + g * 16, 16)]
            pltpu.make_async_copy(bufs[b].at[pl.ds(g * 16, 16)],
                                  acc_sh.at[tv], ssems[b]).wait()

    for ps in range(_NPASS):
        e0 = wid * _EPT + ps * _EPP
        pltpu.sync_copy(src_hbm.at[pl.ds(e0, _EPP)], src_p)
        pltpu.sync_copy(tgt_hbm.at[pl.ds(e0, _EPP)], tgt_p)
        pltpu.sync_copy(vals_hbm.at[pl.ds(e0, _EPP)], vals_p)

        for b in range(_NBUF - 1):
            gather_start(b, b)

        def pbody(p, _):
            for b in range(_NBUF):
                q = _NBUF * p + b
                gather_wait(q, b)
                pb = (b - 1) % _NBUF
                if b == 0:
                    @pl.when(p > 0)
                    def _():
                        scat_drain(q - 1, pb)
                else:
                    scat_drain(q - 1, pb)

                @pl.when(q + _NBUF - 1 < _CPP)
                def _():
                    gather_start(q + _NBUF - 1, pb)

                _scale_chunk(bufs[b], vals_p, q)
                scat_fire(q, b)
            return 0

        ntail = _CPP % _NBUF
        nfull = _CPP // _NBUF
        lax.fori_loop(0, nfull, pbody, 0)
        for t in range(ntail):
            q = nfull * _NBUF + t
            b = q % _NBUF
            gather_wait(q, b)
            scat_drain(q - 1, (b - 1) % _NBUF)
            _scale_chunk(bufs[b], vals_p, q)
            scat_fire(q, b)
        scat_drain(_CPP - 1, (_CPP - 1) % _NBUF)

    plsc.subcore_barrier()

    for j in range(_RPT // _K):
        rr = r0 + j * _K
        pltpu.async_copy(acc_sh.at[pl.ds(rr, _K)],
                         out_hbm.at[c].at[pl.ds(rr, _K)], gsems[j % _NBUF])
    for j in range(_RPT // _K):
        rr = r0 + j * _K
        pltpu.make_async_copy(acc_sh.at[pl.ds(rr, _K)],
                              out_hbm.at[c].at[pl.ds(rr, _K)],
                              gsems[j % _NBUF]).wait()


def _tc_add_body(a_ref, b_ref, o_ref):
    o_ref[...] = a_ref[...] + b_ref[...]


def kernel(x_source, neighborhood_indices, neighborhood_values):
    tgt = neighborhood_indices[0]
    src = neighborhood_indices[1]

    mesh = plsc.VectorSubcoreMesh(core_axis_name="c", subcore_axis_name="s")
    partials = pl.kernel(
        _sc_body,
        mesh=mesh,
        compiler_params=pltpu.CompilerParams(needs_layout_passes=False),
        out_type=jax.ShapeDtypeStruct((_NC, _N_PAD, D_FEAT), jnp.float32),
        scratch_types=[
            pltpu.VMEM_SHARED((_N_PAD, D_FEAT), jnp.float32),
            pltpu.VMEM((_K, D_FEAT), jnp.float32),
            pltpu.VMEM((_K, D_FEAT), jnp.float32),
            pltpu.VMEM((_K, D_FEAT), jnp.float32),
            pltpu.VMEM((_K, D_FEAT), jnp.float32),
            pltpu.VMEM((_EPP,), jnp.int32),
            pltpu.VMEM((_EPP,), jnp.int32),
            pltpu.VMEM((_EPP,), jnp.float32),
            pltpu.SemaphoreType.DMA,
            pltpu.SemaphoreType.DMA,
            pltpu.SemaphoreType.DMA,
            pltpu.SemaphoreType.DMA,
            pltpu.SemaphoreType.DMA,
            pltpu.SemaphoreType.DMA,
            pltpu.SemaphoreType.DMA,
            pltpu.SemaphoreType.DMA,
        ],
    )(x_source, src, tgt, neighborhood_values)

    blk = 1000
    out = pl.pallas_call(
        _tc_add_body,
        out_shape=jax.ShapeDtypeStruct((N_NODES, D_FEAT), jnp.float32),
        grid=(N_NODES // blk,),
        in_specs=[
            pl.BlockSpec((blk, D_FEAT), lambda i: (i, 0)),
            pl.BlockSpec((blk, D_FEAT), lambda i: (i, 0)),
        ],
        out_specs=pl.BlockSpec((blk, D_FEAT), lambda i: (i, 0)),
    )(partials[0], partials[1])
    return out
